# baseline (speedup 1.0000x reference)
import functools

import jax
import jax.numpy as jnp
from jax import lax
from jax.experimental import pallas as pl
from jax.experimental.pallas import tpu as pltpu
from jax.experimental.pallas import tpu_sc as plsc

EMB_DIM = 128
CHARS = 10
NPOS = 10
L = 16
NC, NS = 2, 16
NW = NC * NS
GC = 80
NBUF = 4
NCOL = EMB_DIM // L


def _tc_combine(wt_ref, pt_ref, out_ref):
    wt = wt_ref[...]
    pt = pt_ref[...]
    out_ref[...] = wt[:, None, :] + pt[None, :, :]


def _tc_cidx(tok_ref, pos_ref, out_ref):
    out_ref[...] = (tok_ref[...] * NPOS + pos_ref[...]).T


def _sc_body(W, cidx_hbm, ct_hbm, out_hbm,
             cidx_buf, recip_buf, acc0, acc1, acc2, acc3,
             gsem0, gsem1, gsem2, gsem3, osem0, osem1, osem2, osem3):
    wid = lax.axis_index("s") * NC + lax.axis_index("c")
    wpw = W // NW
    wbase = wid * wpw
    acc = (acc0, acc1, acc2, acc3)
    gsem = (gsem0, gsem1, gsem2, gsem3)
    osem = (osem0, osem1, osem2, osem3)
    nchunks = wpw // GC

    for r in range(CHARS):
        pltpu.sync_copy(cidx_hbm.at[pl.ds(r * W + wbase, wpw)],
                        cidx_buf.at[pl.ds(r * wpw, wpw)])

    @pl.loop(0, wpw // L)
    def count_loop(j):
        cnt = None
        for c in range(CHARS):
            nz = jnp.where(cidx_buf[pl.ds(c * wpw + j * L, L)] >= NPOS,
                           1.0, 0.0)
            cnt = nz if cnt is None else cnt + nz
        recip_buf[pl.ds(j * L, L)] = 1.0 / cnt

    def idx_slice(i, t):
        return cidx_buf.at[pl.ds(t * wpw + i * GC, GC)]

    def prep_and_fire(i, b):
        @pl.loop(0, GC)
        def zero_loop(g):
            for c in range(NCOL):
                acc[b][g, pl.ds(c * L, L)] = jnp.zeros((L,), jnp.float32)

        for t in range(CHARS):
            pltpu.async_copy(ct_hbm.at[idx_slice(i, t)], acc[b], gsem[b],
                             add=True)

    def drain_scale_out(i, b):
        for t in range(CHARS):
            pltpu.make_async_copy(ct_hbm.at[idx_slice(i, t)], acc[b],
                                  gsem[b]).wait()

        @pl.loop(0, GC)
        def scale_loop(g):
            word = i * GC + g
            scale = plsc.load_gather(
                recip_buf, [jnp.broadcast_to(word, (L,)).astype(jnp.int32)])
            for c in range(NCOL):
                sl = pl.ds(c * L, L)
                acc[b][g, sl] = acc[b][g, sl] * scale

        pltpu.async_copy(acc[b], out_hbm.at[pl.ds(wbase + i * GC, GC)],
                         osem[b])

    for k in range(NBUF - 1):
        prep_and_fire(k, k)

    @pl.loop(0, nchunks, step=NBUF)
    def main_loop(i):
        for b in range(NBUF):
            c = i + b
            nslot = (b + NBUF - 1) % NBUF

            @pl.when(c + NBUF - 1 < nchunks)
            def _fire(c=c, nslot=nslot):
                @pl.when(c >= 1)
                def _wait_out(nslot=nslot):
                    pltpu.make_async_copy(
                        acc[nslot],
                        out_hbm.at[pl.ds(wbase, GC)],
                        osem[nslot]).wait()
                prep_and_fire(c + NBUF - 1, nslot)

            drain_scale_out(c, b)

    for b in range(NBUF):
        pltpu.make_async_copy(acc[b], out_hbm.at[pl.ds(wbase, GC)],
                              osem[b]).wait()


def kernel(token_ids, position_ids, word_table, pos_table):
    B, S, C = token_ids.shape
    W = B * S
    tok2d = token_ids.astype(jnp.int32).reshape(W, C)
    pos2d = position_ids.astype(jnp.int32).reshape(W, C)
    blk = 512
    cidx = pl.pallas_call(
        _tc_cidx,
        grid=(W // blk,),
        in_specs=[pl.BlockSpec((blk, C), lambda j: (j, 0)),
                  pl.BlockSpec((blk, C), lambda j: (j, 0))],
        out_specs=pl.BlockSpec((C, blk), lambda j: (0, j)),
        out_shape=jax.ShapeDtypeStruct((C, W), jnp.int32),
    )(tok2d, pos2d).reshape(W * C)

    nv = word_table.shape[0]
    combined = pl.pallas_call(
        _tc_combine,
        out_shape=jax.ShapeDtypeStruct((nv, NPOS, EMB_DIM), jnp.float32),
    )(word_table.astype(jnp.float32), pos_table.astype(jnp.float32))
    combined = combined.reshape(nv * NPOS, EMB_DIM)

    mesh = plsc.VectorSubcoreMesh(core_axis_name="c", subcore_axis_name="s",
                                  num_cores=NC, num_subcores=NS)
    out = pl.kernel(
        functools.partial(_sc_body, W),
        out_type=jax.ShapeDtypeStruct((W, EMB_DIM), jnp.float32),
        mesh=mesh,
        compiler_params=pltpu.CompilerParams(needs_layout_passes=False),
        scratch_types=[
            pltpu.VMEM((CHARS * (W // NW),), jnp.int32),
            pltpu.VMEM((W // NW,), jnp.float32),
            pltpu.VMEM((GC, EMB_DIM), jnp.float32),
            pltpu.VMEM((GC, EMB_DIM), jnp.float32),
            pltpu.VMEM((GC, EMB_DIM), jnp.float32),
            pltpu.VMEM((GC, EMB_DIM), jnp.float32),
            pltpu.SemaphoreType.DMA,
            pltpu.SemaphoreType.DMA,
            pltpu.SemaphoreType.DMA,
            pltpu.SemaphoreType.DMA,
            pltpu.SemaphoreType.DMA,
            pltpu.SemaphoreType.DMA,
            pltpu.SemaphoreType.DMA,
            pltpu.SemaphoreType.DMA,
        ],
    )(cidx, combined)
    return out.reshape(B, S, EMB_DIM)


# speedup vs baseline: 1.4435x; 1.4435x over previous
import functools

import jax
import jax.numpy as jnp
from jax import lax
from jax.experimental import pallas as pl
from jax.experimental.pallas import tpu as pltpu
from jax.experimental.pallas import tpu_sc as plsc

EMB_DIM = 128
CHARS = 10
NPOS = 10
L = 16
NC, NS = 2, 16
NW = NC * NS
GC = 80
NBUF = 4
NCOL = EMB_DIM // L


def _tc_combine(wt_ref, pt_ref, out_ref):
    wt = wt_ref[...]
    pt = pt_ref[...]
    out_ref[...] = wt[:, None, :] + pt[None, :, :]


def _sc_body(W, cidx_hbm, ct_hbm, out_hbm,
             cidx_buf, recip_buf, acc0, acc1, acc2, acc3,
             gsem0, gsem1, gsem2, gsem3, osem0, osem1, osem2, osem3, ssem):
    wid = lax.axis_index("s") * NC + lax.axis_index("c")
    wpw = W // NW
    wbase = wid * wpw
    acc = (acc0, acc1, acc2, acc3)
    gsem = (gsem0, gsem1, gsem2, gsem3)
    osem = (osem0, osem1, osem2, osem3)
    nchunks = wpw // GC

    def stage_desc(r):
        return pltpu.make_async_copy(cidx_hbm.at[pl.ds(r * W + wbase, wpw)],
                                     cidx_buf.at[pl.ds(r * wpw, wpw)], ssem)

    def idx_slice(i, t):
        return cidx_buf.at[pl.ds(t * wpw + i * GC, GC)]

    def zero_acc(b):
        @pl.loop(0, GC)
        def zero_loop(g):
            for c in range(NCOL):
                acc[b][g, pl.ds(c * L, L)] = jnp.zeros((L,), jnp.float32)

    def fire(i, b):
        for t in range(CHARS):
            pltpu.async_copy(ct_hbm.at[idx_slice(i, t)], acc[b], gsem[b],
                             add=True)

    def prep_and_fire(i, b):
        zero_acc(b)
        fire(i, b)

    for r in range(CHARS):
        stage_desc(r).start()
    for b in range(NBUF - 1):
        zero_acc(b)
    for r in range(CHARS):
        stage_desc(r).wait()
    for k in range(NBUF - 1):
        fire(k, k)

    @pl.loop(0, wpw // L)
    def count_loop(j):
        cnt = None
        for c in range(CHARS):
            nz = jnp.where(cidx_buf[pl.ds(c * wpw + j * L, L)] >= NPOS,
                           1.0, 0.0)
            cnt = nz if cnt is None else cnt + nz
        recip_buf[pl.ds(j * L, L)] = 1.0 / cnt

    def drain_scale_out(i, b):
        for t in range(CHARS):
            pltpu.make_async_copy(ct_hbm.at[idx_slice(i, t)], acc[b],
                                  gsem[b]).wait()

        @pl.loop(0, GC)
        def scale_loop(g):
            word = i * GC + g
            scale = plsc.load_gather(
                recip_buf, [jnp.broadcast_to(word, (L,)).astype(jnp.int32)])
            for c in range(NCOL):
                sl = pl.ds(c * L, L)
                acc[b][g, sl] = acc[b][g, sl] * scale

        pltpu.async_copy(acc[b], out_hbm.at[pl.ds(wbase + i * GC, GC)],
                         osem[b])

    @pl.loop(0, nchunks, step=NBUF)
    def main_loop(i):
        for b in range(NBUF):
            c = i + b
            nslot = (b + NBUF - 1) % NBUF

            @pl.when(c + NBUF - 1 < nchunks)
            def _fire(c=c, nslot=nslot):
                @pl.when(c >= 1)
                def _wait_out(nslot=nslot):
                    pltpu.make_async_copy(
                        acc[nslot],
                        out_hbm.at[pl.ds(wbase, GC)],
                        osem[nslot]).wait()
                prep_and_fire(c + NBUF - 1, nslot)

            drain_scale_out(c, b)

    for b in range(NBUF):
        pltpu.make_async_copy(acc[b], out_hbm.at[pl.ds(wbase, GC)],
                              osem[b]).wait()


def kernel(token_ids, position_ids, word_table, pos_table):
    B, S, C = token_ids.shape
    W = B * S
    tok2d = token_ids.astype(jnp.int32).reshape(W, C)
    pos2d = position_ids.astype(jnp.int32).reshape(W, C)
    cidx = (tok2d * NPOS + pos2d).T.reshape(W * C)

    nv = word_table.shape[0]
    combined = pl.pallas_call(
        _tc_combine,
        out_shape=jax.ShapeDtypeStruct((nv, NPOS, EMB_DIM), jnp.float32),
    )(word_table.astype(jnp.float32), pos_table.astype(jnp.float32))
    combined = combined.reshape(nv * NPOS, EMB_DIM)

    mesh = plsc.VectorSubcoreMesh(core_axis_name="c", subcore_axis_name="s",
                                  num_cores=NC, num_subcores=NS)
    out = pl.kernel(
        functools.partial(_sc_body, W),
        out_type=jax.ShapeDtypeStruct((W, EMB_DIM), jnp.float32),
        mesh=mesh,
        compiler_params=pltpu.CompilerParams(needs_layout_passes=False),
        scratch_types=[
            pltpu.VMEM((CHARS * (W // NW),), jnp.int32),
            pltpu.VMEM((W // NW,), jnp.float32),
            pltpu.VMEM((GC, EMB_DIM), jnp.float32),
            pltpu.VMEM((GC, EMB_DIM), jnp.float32),
            pltpu.VMEM((GC, EMB_DIM), jnp.float32),
            pltpu.VMEM((GC, EMB_DIM), jnp.float32),
            pltpu.SemaphoreType.DMA,
            pltpu.SemaphoreType.DMA,
            pltpu.SemaphoreType.DMA,
            pltpu.SemaphoreType.DMA,
            pltpu.SemaphoreType.DMA,
            pltpu.SemaphoreType.DMA,
            pltpu.SemaphoreType.DMA,
            pltpu.SemaphoreType.DMA,
            pltpu.SemaphoreType.DMA,
        ],
    )(cidx, combined)
    return out.reshape(B, S, EMB_DIM)
